# baseline (speedup 1.0000x reference)
import jax
import jax.numpy as jnp
from jax import lax
from jax.experimental import pallas as pl
from jax.experimental.pallas import tpu as pltpu
from jax.experimental.pallas import tpu_sc as plsc

B, F, P = 1024, 26, 40
VOCAB, EMB = 100000, 32

N = B * F * P
NC, NS = 2, 16
NW = NC * NS
CH = 1280
G = 26


def _sc_write(table):
    mesh = plsc.VectorSubcoreMesh(core_axis_name="c", subcore_axis_name="s")

    @pl.kernel(
        out_type=jax.ShapeDtypeStruct((N, EMB), jnp.float32),
        mesh=mesh,
        scratch_types=[
            pltpu.VMEM((CH, EMB), jnp.float32),
            pltpu.VMEM_SHARED((NS * CH, EMB), jnp.float32),
            pltpu.SemaphoreType.DMA,
            pltpu.SemaphoreType.DMA,
        ],
        compiler_params=pltpu.CompilerParams(use_tc_tiling_on_sc=False),
    )
    def k(table_hbm, out_hbm, rows0, spbuf, sem, osem):
        cid = lax.axis_index("c")
        sid = lax.axis_index("s")

        pltpu.sync_copy(table_hbm.at[pl.ds(0, CH)], rows0)

        def body(t, _):
            pltpu.sync_copy(rows0, spbuf.at[pl.ds(sid * CH, CH)])
            plsc.subcore_barrier()

            @pl.when(sid == 0)
            def _flush():
                pltpu.sync_copy(
                    spbuf,
                    out_hbm.at[pl.ds((cid * G + t) * (NS * CH), NS * CH)])
            plsc.subcore_barrier()
            return _

        lax.fori_loop(0, G, body, None)

    return k(table)


def kernel(feature, table):
    out = _sc_write(table)
    return out.reshape(B, F, P * EMB)


# speedup vs baseline: 1.0525x; 1.0525x over previous
import jax
import jax.numpy as jnp
from jax import lax
from jax.experimental import pallas as pl
from jax.experimental.pallas import tpu as pltpu
from jax.experimental.pallas import tpu_sc as plsc

B, F, P = 1024, 26, 40
VOCAB, EMB = 100000, 32

N = B * F * P
NC, NS = 2, 16
NW = NC * NS
CH = 640
NPW = N // NW
CPW = NPW // CH
NBUF = 4
GB = CPW // NBUF


def _sc_gather(table, idx):
    mesh = plsc.VectorSubcoreMesh(core_axis_name="c", subcore_axis_name="s")

    @pl.kernel(
        out_type=jax.ShapeDtypeStruct((N, EMB), jnp.float32),
        mesh=mesh,
        scratch_types=[
            pltpu.VMEM((CPW, CH), jnp.int32),
            [pltpu.VMEM((CH, EMB), jnp.float32) for _ in range(NBUF)],
            pltpu.SemaphoreType.DMA,
            [pltpu.SemaphoreType.DMA for _ in range(NBUF)],
            [pltpu.SemaphoreType.DMA for _ in range(NBUF)],
        ],
        compiler_params=pltpu.CompilerParams(use_tc_tiling_on_sc=False),
    )
    def k(table_hbm, idx_hbm, out_hbm, idx_v, rows, isem, gsem, ssem):
        wid = lax.axis_index("s") * NC + lax.axis_index("c")
        cbase = wid * CPW

        pltpu.sync_copy(idx_hbm.at[wid], idx_v)

        def gather(t, b):
            return pltpu.make_async_copy(
                table_hbm.at[idx_v.at[t]], rows[b], gsem[b])

        def store(t, b):
            return pltpu.make_async_copy(
                rows[b], out_hbm.at[pl.ds((cbase + t) * CH, CH)], ssem[b])

        def body(u, _):
            t0 = u * NBUF

            @pl.when(u > 0)
            def _drain():
                for b in range(NBUF):
                    store(t0 + b, b).wait()

            for b in range(NBUF):
                gather(t0 + b, b).start()
            for b in range(NBUF):
                gather(t0 + b, b).wait()
                store(t0 + b, b).start()
            return _

        lax.fori_loop(0, GB, body, None)
        for b in range(NBUF):
            store(b, b).wait()

    return k(table, idx)


def kernel(feature, table):
    idx = feature.reshape(NW, CPW, CH)
    out = _sc_gather(table, idx)
    return out.reshape(B, F, P * EMB)


# speedup vs baseline: 1.0533x; 1.0008x over previous
import jax
import jax.numpy as jnp
from jax import lax
from jax.experimental import pallas as pl
from jax.experimental.pallas import tpu as pltpu
from jax.experimental.pallas import tpu_sc as plsc

B, F, P = 1024, 26, 40
VOCAB, EMB = 100000, 32

N = B * F * P
NC, NS = 2, 16
NW = NC * NS
CH = 640
NPW = N // NW
CPW = NPW // CH
NBUF = 4
GB = CPW // NBUF


def _sc_gather(table, idx):
    mesh = plsc.VectorSubcoreMesh(core_axis_name="c", subcore_axis_name="s")

    @pl.kernel(
        out_type=jax.ShapeDtypeStruct((N, EMB), jnp.float32),
        mesh=mesh,
        scratch_types=[
            pltpu.VMEM((NPW,), jnp.int32),
            [pltpu.VMEM((CH, EMB), jnp.float32) for _ in range(NBUF)],
            pltpu.SemaphoreType.DMA,
            [pltpu.SemaphoreType.DMA for _ in range(NBUF)],
            [pltpu.SemaphoreType.DMA for _ in range(NBUF)],
        ],
        compiler_params=pltpu.CompilerParams(use_tc_tiling_on_sc=False),
    )
    def k(table_hbm, idx_hbm, out_hbm, idx_v, rows, isem, gsem, ssem):
        wid = lax.axis_index("s") * NC + lax.axis_index("c")
        cbase = wid * CPW

        pltpu.sync_copy(idx_hbm.at[pl.ds(wid * NPW, NPW)], idx_v)

        def gather(t, b):
            return pltpu.make_async_copy(
                table_hbm.at[idx_v.at[pl.ds(t * CH, CH)]], rows[b], gsem[b])

        def store(t, b):
            return pltpu.make_async_copy(
                rows[b], out_hbm.at[pl.ds((cbase + t) * CH, CH)], ssem[b])

        def body(u, _):
            t0 = u * NBUF

            @pl.when(u > 0)
            def _drain():
                for b in range(NBUF):
                    store(t0 + b, b).wait()

            for b in range(NBUF):
                gather(t0 + b, b).start()
            for b in range(NBUF):
                gather(t0 + b, b).wait()
                store(t0 + b, b).start()
            return _

        lax.fori_loop(0, GB, body, None)
        for b in range(NBUF):
            store(b, b).wait()

    return k(table, idx)


def kernel(feature, table):
    idx = feature.reshape(N)
    out = _sc_gather(table, idx)
    return out.reshape(B, F, P * EMB)


# speedup vs baseline: 1.2188x; 1.1571x over previous
import jax
import jax.numpy as jnp
from jax import lax
from jax.experimental import pallas as pl
from jax.experimental.pallas import tpu as pltpu
from jax.experimental.pallas import tpu_sc as plsc

B, F, P = 1024, 26, 40
VOCAB, EMB = 100000, 32

N = B * F * P
NC, NS = 2, 16
NW = NC * NS
BW = B // NW
FP = F * P
PG = 8
NT = FP // PG
NBUF = 2


def _sc_gather(table, idxfp):
    mesh = plsc.VectorSubcoreMesh(core_axis_name="c", subcore_axis_name="s")

    @pl.kernel(
        out_type=jax.ShapeDtypeStruct((F * B, P * EMB), jnp.float32),
        mesh=mesh,
        scratch_types=[
            pltpu.VMEM((FP, BW), jnp.int32),
            [pltpu.VMEM((PG, BW, EMB), jnp.float32) for _ in range(NBUF)],
            [pltpu.SemaphoreType.DMA for _ in range(NBUF)],
            [pltpu.SemaphoreType.DMA for _ in range(NBUF)],
        ],
        compiler_params=pltpu.CompilerParams(use_tc_tiling_on_sc=False),
    )
    def k(table_hbm, idx_hbm, out_hbm, idx_v, rows, gsem, ssem):
        wid = lax.axis_index("s") * NC + lax.axis_index("c")
        b0 = wid * BW

        pltpu.sync_copy(idx_hbm.at[pl.ds(0, FP), pl.ds(b0, BW)], idx_v)

        def gathers(t, b):
            return [
                pltpu.make_async_copy(table_hbm.at[idx_v.at[t * PG + q]],
                                      rows[b].at[q], gsem[b])
                for q in range(PG)
            ]

        def stores(t, b):
            f = t // (P // PG)
            p0 = (t % (P // PG)) * PG
            return [
                pltpu.make_async_copy(
                    rows[b].at[q],
                    out_hbm.at[pl.ds(f * B + b0, BW),
                               pl.ds((p0 + q) * EMB, EMB)],
                    ssem[b])
                for q in range(PG)
            ]

        def body(u, _):
            for b in range(NBUF):
                t = u * NBUF + b

                @pl.when(u > 0)
                def _drain():
                    for c in stores(t, b):
                        c.wait()

                g = gathers(t, b)
                for c in g:
                    c.start()
                for c in g:
                    c.wait()
                for c in stores(t, b):
                    c.start()
            return _

        lax.fori_loop(0, NT // NBUF, body, None)
        for b in range(NBUF):
            for c in stores(b, b):
                c.wait()

    return k(table, idxfp)


def kernel(feature, table):
    idxfp = feature.transpose(1, 2, 0).reshape(FP, B)
    out = _sc_gather(table, idxfp)
    return out.reshape(F, B, P * EMB).transpose(1, 0, 2)


# speedup vs baseline: 1.4910x; 1.2233x over previous
import jax
import jax.numpy as jnp
from jax import lax
from jax.experimental import pallas as pl
from jax.experimental.pallas import tpu as pltpu
from jax.experimental.pallas import tpu_sc as plsc

B, F, P = 1024, 26, 40
VOCAB, EMB = 100000, 32

N = B * F * P
NC, NS = 2, 16
NW = NC * NS
BW = B // NW
FP = F * P
PB = P * BW
NBUF = 2


def _sc_gather(table, idxfp):
    mesh = plsc.VectorSubcoreMesh(core_axis_name="c", subcore_axis_name="s")

    @pl.kernel(
        out_type=jax.ShapeDtypeStruct((N, EMB), jnp.float32),
        mesh=mesh,
        scratch_types=[
            pltpu.VMEM((FP, BW), jnp.int32),
            [pltpu.VMEM((PB,), jnp.int32) for _ in range(NBUF)],
            [pltpu.VMEM((PB, EMB), jnp.float32) for _ in range(NBUF)],
            [pltpu.SemaphoreType.DMA for _ in range(NBUF)],
            [pltpu.SemaphoreType.DMA for _ in range(NBUF)],
        ],
        compiler_params=pltpu.CompilerParams(use_tc_tiling_on_sc=False,
                                             needs_layout_passes=False),
    )
    def k(table_hbm, idx_hbm, out_hbm, idx_v, idx_t, rows, gsem, ssem):
        wid = lax.axis_index("s") * NC + lax.axis_index("c")
        b0 = wid * BW

        pltpu.sync_copy(idx_hbm.at[pl.ds(0, FP), pl.ds(b0, BW)], idx_v)

        iota = lax.iota(jnp.int32, 16)

        def permute(f, b):
            def vec(j, _):
                kv = iota + j * 16
                pv = lax.rem(kv, P)
                bv = lax.div(kv, P)
                g = plsc.load_gather(idx_v, [pv + f * P, bv])
                idx_t[b][pl.ds(j * 16, 16)] = g
                return _
            lax.fori_loop(0, PB // 16, vec, None)

        def gather(f, b):
            return pltpu.make_async_copy(table_hbm.at[idx_t[b]], rows[b],
                                         gsem[b])

        def store(f, b):
            return pltpu.make_async_copy(
                rows[b], out_hbm.at[pl.ds((f * B + b0) * P, PB)], ssem[b])

        def body(u, _):
            for b in range(NBUF):
                f = u * NBUF + b

                @pl.when(u > 0)
                def _drain():
                    store(f, b).wait()

                permute(f, b)
                gather(f, b).start()
            for b in range(NBUF):
                f = u * NBUF + b
                gather(f, b).wait()
                store(f, b).start()
            return _

        lax.fori_loop(0, F // NBUF, body, None)
        for b in range(NBUF):
            store(b, b).wait()

    return k(table, idxfp)


def kernel(feature, table):
    idxfp = feature.transpose(1, 2, 0).reshape(FP, B)
    out = _sc_gather(table, idxfp)
    return out.reshape(F, B, P * EMB).transpose(1, 0, 2)
